# baseline (speedup 1.0000x reference)
import functools

import jax
import jax.numpy as jnp
from jax import lax
from jax.experimental import pallas as pl
from jax.experimental.pallas import tpu as pltpu
from jax.experimental.pallas import tpu_sc as plsc

B = 16384
D = 64
D2 = 2 * D
HALF = 32
NC, NS = 2, 16
NW = NC * NS
B_PER_W = B // NW
CHUNK = 128
NCHUNK = B_PER_W // CHUNK

_mesh = plsc.VectorSubcoreMesh(core_axis_name="c", subcore_axis_name="s")


@functools.partial(
    pl.kernel,
    mesh=_mesh,
    out_type=jax.ShapeDtypeStruct((B, D2), jnp.float32),
    scratch_types=[
        pltpu.VMEM((NCHUNK, CHUNK), jnp.int32),
        pltpu.VMEM((B_PER_W, D2), jnp.float32),
        pltpu.SemaphoreType.DMA,
    ],
    compiler_params=pltpu.CompilerParams(use_tc_tiling_on_sc=True),
)
def _sc_gather(idx_hbm, table_hbm, out_hbm, idx_v, rows_v, sem):
    wid = lax.axis_index("s") * NC + lax.axis_index("c")
    pltpu.sync_copy(idx_hbm.at[wid], idx_v)
    copies = [
        pltpu.async_copy(
            table_hbm.at[idx_v.at[j]],
            rows_v.at[pl.ds(j * CHUNK, CHUNK)],
            sem,
        )
        for j in range(NCHUNK)
    ]
    for c in copies:
        c.wait()
    pltpu.sync_copy(rows_v, out_hbm.at[pl.ds(wid * B_PER_W, B_PER_W)])


BLK = 2048
GRID = B // BLK


def _tc_body(g_ref, par_ref, bkt_ref, wbT_ref, bb_ref, weT_ref, wb2T_ref,
             bc_ref, out_ref):
    g = g_ref[...]
    odd = par_ref[...] > 0
    emb = jnp.where(odd, g[:, D:], g[:, :D])
    f = jnp.dot(bkt_ref[...], wbT_ref[...], preferred_element_type=jnp.float32)
    f = f + bb_ref[...]
    y = jnp.dot(emb, weT_ref[...], preferred_element_type=jnp.float32)
    y = y + jnp.dot(f, wb2T_ref[...], preferred_element_type=jnp.float32)
    out_ref[...] = jnp.tanh(y + bc_ref[...])


_tc_dense = pl.pallas_call(
    _tc_body,
    grid=(GRID,),
    in_specs=[
        pl.BlockSpec((BLK, D2), lambda i: (i, 0)),
        pl.BlockSpec((BLK, 1), lambda i: (i, 0)),
        pl.BlockSpec((BLK, 4), lambda i: (i, 0)),
        pl.BlockSpec((4, HALF), lambda i: (0, 0)),
        pl.BlockSpec((1, HALF), lambda i: (0, 0)),
        pl.BlockSpec((D, D), lambda i: (0, 0)),
        pl.BlockSpec((HALF, D), lambda i: (0, 0)),
        pl.BlockSpec((1, D), lambda i: (0, 0)),
    ],
    out_specs=pl.BlockSpec((BLK, D), lambda i: (i, 0)),
    out_shape=jax.ShapeDtypeStruct((B, D), jnp.float32),
)


def kernel(skill_ids, bkt_params, table, W_bkt, b_bkt, W_comb, b_comb):
    ids = skill_ids.astype(jnp.int32)
    pair_idx = (ids >> 1).reshape(NW, NCHUNK, CHUNK)
    parity = (ids & 1).reshape(B, 1)
    table_pairs = table.reshape(table.shape[0] // 2, D2)
    gathered = _sc_gather(pair_idx, table_pairs)
    return _tc_dense(
        gathered,
        parity,
        bkt_params,
        W_bkt.T,
        b_bkt.reshape(1, HALF),
        W_comb[:, :D].T,
        W_comb[:, D:].T,
        b_comb.reshape(1, D),
    )


# speedup vs baseline: 1.3593x; 1.3593x over previous
import functools

import jax
import jax.numpy as jnp
from jax import lax
from jax.experimental import pallas as pl
from jax.experimental.pallas import tpu as pltpu
from jax.experimental.pallas import tpu_sc as plsc

B = 16384
D = 64
HALF = 32
NC, NS = 2, 16
NW = NC * NS
V = 1000000

C = 512
NFULL = 1953
TAIL_START = NFULL * C
TAIL_W = V - TAIL_START
RCAP = 256
DUMP = B
OUTROWS = B + 8
SENT = 0x7FFFFFFF

_mesh = plsc.VectorSubcoreMesh(core_axis_name="c", subcore_axis_name="s")


@functools.partial(
    pl.kernel,
    mesh=_mesh,
    out_type=jax.ShapeDtypeStruct((OUTROWS, 128), jnp.float32),
    scratch_types=[
        pltpu.VMEM((4096,), jnp.int32),
        pltpu.VMEM((16400,), jnp.int32),
        pltpu.VMEM((16400,), jnp.int32),
        pltpu.VMEM((D, C), jnp.float32),
        pltpu.VMEM((D, TAIL_W), jnp.float32),
        pltpu.VMEM((RCAP, 128), jnp.float32),
        pltpu.VMEM((RCAP // 16, 16), jnp.int32),
        pltpu.SemaphoreType.DMA,
    ],
    compiler_params=pltpu.CompilerParams(use_tc_tiling_on_sc=True,
                                         needs_layout_passes=False),
)
def _sc_stream_select(ids_hbm, table_t_hbm, tail_hbm, out_hbm, idsbuf, idlist,
                      cwork, slab, tailslab, staging, plist, sem):
    wid = lax.axis_index("s") * NC + lax.axis_index("c")
    iota = lax.iota(jnp.int32, 16)
    sentv = jnp.full((16,), SENT, jnp.int32)
    dumpv = jnp.full((16,), jnp.int32(DUMP), jnp.int32)

    def reset_plist():
        for h in range(RCAP // 16):
            plist[h] = dumpv

    reset_plist()

    @pl.loop(0, 16400 // 16)
    def _(i):
        idlist[pl.ds(i * 16, 16)] = sentv

    off = jnp.int32(0)
    for p in range(4):
        pltpu.sync_copy(ids_hbm.at[pl.ds(p * 4096, 4096)], idsbuf)

        @pl.loop(0, 256, init_carry=off)
        def _collect(i, acc, _p=p):
            v = idsbuf[pl.ds(i * 16, 16)]
            m = ((v >> 9) & 31) == wid
            pos = iota + (_p * 4096 + i * 16)
            packed = (pos << 15) | ((v >> 14) << 9) | (v & 511)
            plsc.store_compressed(idlist.at[pl.ds(acc, 16)], packed, mask=m)
            return acc + jnp.sum(m.astype(jnp.int32))

        off = _collect

    n_w = off
    nv = (n_w + 15) >> 4

    def flush_effects():
        copies = [
            pltpu.async_copy(staging.at[pl.ds(h * 16, 16)],
                             out_hbm.at[plist.at[h]], sem)
            for h in range(RCAP // 16)
        ]
        for cph in copies:
            cph.wait()
        reset_plist()

    def process_chunk(ck, slot, sl):
        @pl.loop(0, nv, init_carry=jnp.int32(0))
        def _scan(i, n2):
            e = idlist[pl.ds(i * 16, 16)]
            m2 = ((e >> 9) & 63) == ck
            plsc.store_compressed(cwork.at[pl.ds(n2, 16)], e, mask=m2)
            return n2 + jnp.sum(m2.astype(jnp.int32))

        n2 = _scan
        ngrp = (n2 + 15) >> 4

        @pl.loop(0, ngrp, init_carry=slot)
        def _grp(g, s):
            ev = cwork[pl.ds(g * 16, 16)]
            full = s > jnp.int32(RCAP - 16)

            @pl.when(full)
            def _():
                flush_effects()

            s = jnp.where(full, jnp.int32(0), s)
            cnt = jnp.minimum(jnp.int32(16), n2 - g * 16)

            @pl.loop(0, cnt, init_carry=s)
            def _one(i, s2):
                lane = (iota == i)
                e = jnp.sum(jnp.where(lane, ev, 0))
                cc = e & 511
                pos = e >> 15
                ccv = jnp.full((16,), cc, jnp.int32)
                for j in range(4):
                    vals = plsc.load_gather(sl, [iota + 16 * j, ccv])
                    staging[s2, pl.ds(16 * j, 16)] = vals
                row = s2 >> 4
                prow = plist[row]
                plist[row] = jnp.where(iota == (s2 & 15), pos, prow)
                return s2 + 1

            return _one

        return _grp

    nfull = jnp.where(wid == 0, jnp.int32(62), jnp.int32(61))

    @pl.loop(0, nfull, init_carry=jnp.int32(0))
    def _main(k, slot):
        start = (k * NW + wid) * C
        pltpu.sync_copy(table_t_hbm.at[:, pl.ds(start, C)], slab)
        return process_chunk(k, slot, slab)

    def _tail(slot):
        pltpu.sync_copy(tail_hbm, tailslab)
        return process_chunk(jnp.int32(61), slot, tailslab)

    slot = lax.cond(wid == 1, _tail, lambda s: s, _main)
    del slot
    flush_effects()


BLK = 2048
GRID = B // BLK


def _tc_body(g_ref, bktt_ref, wbkt_ref, bb_ref, we_ref, wb2_ref, bc_ref,
             out_ref):
    f_t = jnp.dot(wbkt_ref[...], bktt_ref[...],
                  preferred_element_type=jnp.float32) + bb_ref[...]
    emb = g_ref[...][:, :D]
    y = lax.dot_general(we_ref[...], emb, (((1,), (1,)), ((), ())),
                        preferred_element_type=jnp.float32)
    y = y + jnp.dot(wb2_ref[...], f_t, preferred_element_type=jnp.float32)
    out_ref[...] = jnp.tanh(y + bc_ref[...])


_tc_dense = pl.pallas_call(
    _tc_body,
    grid=(GRID,),
    in_specs=[
        pl.BlockSpec((BLK, 128), lambda i: (i, 0)),
        pl.BlockSpec((4, BLK), lambda i: (0, i)),
        pl.BlockSpec((HALF, 4), lambda i: (0, 0)),
        pl.BlockSpec((HALF, 1), lambda i: (0, 0)),
        pl.BlockSpec((D, D), lambda i: (0, 0)),
        pl.BlockSpec((D, HALF), lambda i: (0, 0)),
        pl.BlockSpec((D, 1), lambda i: (0, 0)),
    ],
    out_specs=pl.BlockSpec((D, BLK), lambda i: (0, i)),
    out_shape=jax.ShapeDtypeStruct((D, B), jnp.float32),
)


def kernel(skill_ids, bkt_params, table, W_bkt, b_bkt, W_comb, b_comb):
    ids = skill_ids.astype(jnp.int32)
    table_t = table.T
    gathered_pad = _sc_stream_select(ids, table_t,
                                     table_t[:, TAIL_START:])
    out_t = _tc_dense(
        gathered_pad,
        bkt_params.T,
        W_bkt,
        b_bkt.reshape(HALF, 1),
        W_comb[:, :D],
        W_comb[:, D:],
        b_comb.reshape(D, 1),
    )
    return out_t.T


# speedup vs baseline: 1.8742x; 1.3788x over previous
import functools

import jax
import jax.numpy as jnp
from jax import lax
from jax.experimental import pallas as pl
from jax.experimental.pallas import tpu as pltpu
from jax.experimental.pallas import tpu_sc as plsc

B = 16384
D = 64
HALF = 32
NC, NS = 2, 16
NW = NC * NS
V = 1000000

C = 512
NFULL = 1953
TAIL_START = NFULL * C
TAIL_W = V - TAIL_START
RCAP = 128
DUMP = B
OUTROWS = B + 8
SENT = 0x7FFFFFFF

_mesh = plsc.VectorSubcoreMesh(core_axis_name="c", subcore_axis_name="s")


@functools.partial(
    pl.kernel,
    mesh=_mesh,
    out_type=jax.ShapeDtypeStruct((OUTROWS, 128), jnp.float32),
    scratch_types=[
        pltpu.VMEM((4096,), jnp.int32),
        pltpu.VMEM((16400,), jnp.int32),
        pltpu.VMEM((16400,), jnp.int32),
        pltpu.VMEM((D, C), jnp.float32),
        pltpu.VMEM((D, C), jnp.float32),
        pltpu.VMEM((D, TAIL_W), jnp.float32),
        pltpu.VMEM((RCAP, 128), jnp.float32),
        pltpu.VMEM((RCAP // 16, 16), jnp.int32),
        pltpu.SemaphoreType.DMA,
        pltpu.SemaphoreType.DMA,
        pltpu.SemaphoreType.DMA,
    ],
    compiler_params=pltpu.CompilerParams(use_tc_tiling_on_sc=True,
                                         needs_layout_passes=False),
)
def _sc_stream_select(ids_hbm, table_t_hbm, tail_hbm, out_hbm, idsbuf, idlist,
                      cwork, slabA, slabB, tailslab, staging, plist, sem,
                      semA, semB):
    wid = lax.axis_index("s") * NC + lax.axis_index("c")
    iota = lax.iota(jnp.int32, 16)
    sentv = jnp.full((16,), SENT, jnp.int32)
    dumpv = jnp.full((16,), jnp.int32(DUMP), jnp.int32)

    def reset_plist():
        for h in range(RCAP // 16):
            plist[h] = dumpv

    reset_plist()

    @pl.loop(0, 16400 // 16)
    def _(i):
        idlist[pl.ds(i * 16, 16)] = sentv

    off = jnp.int32(0)
    for p in range(4):
        pltpu.sync_copy(ids_hbm.at[pl.ds(p * 4096, 4096)], idsbuf)

        @pl.loop(0, 256, init_carry=off)
        def _collect(i, acc, _p=p):
            v = idsbuf[pl.ds(i * 16, 16)]
            m = ((v >> 9) & 31) == wid
            pos = iota + (_p * 4096 + i * 16)
            packed = (pos << 15) | ((v >> 14) << 9) | (v & 511)
            plsc.store_compressed(idlist.at[pl.ds(acc, 16)], packed, mask=m)
            return acc + plsc.all_reduce_population_count(m)[0]

        off = _collect

    n_w = off
    nv = (n_w + 15) >> 4

    def flush_effects():
        copies = [
            pltpu.async_copy(staging.at[pl.ds(h * 16, 16)],
                             out_hbm.at[plist.at[h]], sem)
            for h in range(RCAP // 16)
        ]
        for cph in copies:
            cph.wait()
        reset_plist()

    def process_chunk(ck, slot, sl):
        @pl.loop(0, nv, init_carry=jnp.int32(0))
        def _scan(i, n2):
            e = idlist[pl.ds(i * 16, 16)]
            m2 = ((e >> 9) & 63) == ck
            plsc.store_compressed(cwork.at[pl.ds(n2, 16)], e, mask=m2)
            return n2 + plsc.all_reduce_population_count(m2)[0]

        n2 = _scan
        ngrp = (n2 + 15) >> 4

        @pl.loop(0, ngrp, init_carry=slot)
        def _grp(g, s):
            ev = cwork[pl.ds(g * 16, 16)]
            full = s > jnp.int32(RCAP - 16)

            @pl.when(full)
            def _():
                flush_effects()

            s = jnp.where(full, jnp.int32(0), s)
            cnt = jnp.minimum(jnp.int32(16), n2 - g * 16)

            @pl.loop(0, cnt, init_carry=s)
            def _one(i, s2):
                lane = (iota == i)
                e = jnp.sum(jnp.where(lane, ev, 0))
                cc = e & 511
                pos = e >> 15
                ccv = jnp.full((16,), cc, jnp.int32)
                for j in range(4):
                    vals = plsc.load_gather(sl, [iota + 16 * j, ccv])
                    staging[s2, pl.ds(16 * j, 16)] = vals
                row = s2 >> 4
                prow = plist[row]
                plist[row] = jnp.where(iota == (s2 & 15), pos, prow)
                return s2 + 1

            return _one

        return _grp

    def issue(k, sl, sm):
        start = jnp.minimum((k * NW + wid) * C, jnp.int32((NFULL - 1) * C))
        pltpu.async_copy(table_t_hbm.at[:, pl.ds(start, C)], sl, sm)

    def wait_slab(sl, sm):
        pltpu.make_async_copy(table_t_hbm.at[:, pl.ds(0, C)], sl, sm).wait()

    def ck_eff(k):
        return jnp.where(k * NW + wid <= NFULL - 1, k, jnp.int32(63))

    issue(jnp.int32(0), slabA, semA)

    @pl.loop(0, 31, init_carry=jnp.int32(0))
    def _main(k2, s):
        k = k2 * 2
        issue(k + 1, slabB, semB)
        wait_slab(slabA, semA)
        s = process_chunk(ck_eff(k), s, slabA)
        issue(k + 2, slabA, semA)
        wait_slab(slabB, semB)
        s = process_chunk(ck_eff(k + 1), s, slabB)
        return s

    wait_slab(slabA, semA)

    def _tail(slot):
        pltpu.sync_copy(tail_hbm, tailslab)
        return process_chunk(jnp.int32(61), slot, tailslab)

    slot = lax.cond(wid == 1, _tail, lambda s: s, _main)
    del slot
    flush_effects()


BLK = 2048
GRID = B // BLK


def _tc_body(g_ref, bktt_ref, wbkt_ref, bb_ref, we_ref, wb2_ref, bc_ref,
             out_ref):
    f_t = jnp.dot(wbkt_ref[...], bktt_ref[...],
                  preferred_element_type=jnp.float32) + bb_ref[...]
    emb = g_ref[...][:, :D]
    y = lax.dot_general(we_ref[...], emb, (((1,), (1,)), ((), ())),
                        preferred_element_type=jnp.float32)
    y = y + jnp.dot(wb2_ref[...], f_t, preferred_element_type=jnp.float32)
    out_ref[...] = jnp.tanh(y + bc_ref[...])


_tc_dense = pl.pallas_call(
    _tc_body,
    grid=(GRID,),
    in_specs=[
        pl.BlockSpec((BLK, 128), lambda i: (i, 0)),
        pl.BlockSpec((4, BLK), lambda i: (0, i)),
        pl.BlockSpec((HALF, 4), lambda i: (0, 0)),
        pl.BlockSpec((HALF, 1), lambda i: (0, 0)),
        pl.BlockSpec((D, D), lambda i: (0, 0)),
        pl.BlockSpec((D, HALF), lambda i: (0, 0)),
        pl.BlockSpec((D, 1), lambda i: (0, 0)),
    ],
    out_specs=pl.BlockSpec((D, BLK), lambda i: (0, i)),
    out_shape=jax.ShapeDtypeStruct((D, B), jnp.float32),
)


def kernel(skill_ids, bkt_params, table, W_bkt, b_bkt, W_comb, b_comb):
    ids = skill_ids.astype(jnp.int32)
    table_t = table.T
    gathered_pad = _sc_stream_select(ids, table_t,
                                     table_t[:, TAIL_START:])
    out_t = _tc_dense(
        gathered_pad,
        bkt_params.T,
        W_bkt,
        b_bkt.reshape(HALF, 1),
        W_comb[:, :D],
        W_comb[:, D:],
        b_comb.reshape(D, 1),
    )
    return out_t.T


# speedup vs baseline: 1.8755x; 1.0007x over previous
import functools

import jax
import jax.numpy as jnp
from jax import lax
from jax.experimental import pallas as pl
from jax.experimental.pallas import tpu as pltpu
from jax.experimental.pallas import tpu_sc as plsc

B = 16384
D = 64
HALF = 32
NC, NS = 2, 16
NW = NC * NS
V = 1000000

C = 512
NFULL = 1953
TAIL_START = NFULL * C
TAIL_W = V - TAIL_START
RCAP = 128
DUMP = B
OUTROWS = B + 8
SENT = 0x7FFFFFFF

_mesh = plsc.VectorSubcoreMesh(core_axis_name="c", subcore_axis_name="s")


@functools.partial(
    pl.kernel,
    mesh=_mesh,
    out_type=jax.ShapeDtypeStruct((OUTROWS, 128), jnp.float32),
    scratch_types=[
        pltpu.VMEM((4096,), jnp.int32),
        pltpu.VMEM((16400,), jnp.int32),
        pltpu.VMEM((16400,), jnp.int32),
        pltpu.VMEM((D, C), jnp.float32),
        pltpu.VMEM((D, C), jnp.float32),
        pltpu.VMEM((D, TAIL_W), jnp.float32),
        pltpu.VMEM((RCAP, 128), jnp.float32),
        pltpu.VMEM((RCAP // 16, 16), jnp.int32),
        pltpu.SemaphoreType.DMA,
        pltpu.SemaphoreType.DMA,
        pltpu.SemaphoreType.DMA,
    ],
    compiler_params=pltpu.CompilerParams(use_tc_tiling_on_sc=True,
                                         needs_layout_passes=False),
)
def _sc_stream_select(ids_hbm, table_t_hbm, tail_hbm, out_hbm, idsbuf, idlist,
                      cwork, slabA, slabB, tailslab, staging, plist, sem,
                      semA, semB):
    wid = lax.axis_index("s") * NC + lax.axis_index("c")
    iota = lax.iota(jnp.int32, 16)
    sentv = jnp.full((16,), SENT, jnp.int32)
    dumpv = jnp.full((16,), jnp.int32(DUMP), jnp.int32)

    def reset_plist():
        for h in range(RCAP // 16):
            plist[h] = dumpv

    reset_plist()

    @pl.loop(0, 16400 // 16)
    def _(i):
        idlist[pl.ds(i * 16, 16)] = sentv

    off = jnp.int32(0)
    for p in range(4):
        pltpu.sync_copy(ids_hbm.at[pl.ds(p * 4096, 4096)], idsbuf)

        @pl.loop(0, 256, init_carry=off)
        def _collect(i, acc, _p=p):
            v = idsbuf[pl.ds(i * 16, 16)]
            m = ((v >> 9) & 31) == wid
            pos = iota + (_p * 4096 + i * 16)
            packed = (pos << 15) | ((v >> 14) << 9) | (v & 511)
            plsc.store_compressed(idlist.at[pl.ds(acc, 16)], packed, mask=m)
            return acc + plsc.all_reduce_population_count(m)[0]

        off = _collect

    n_w = off
    nv = (n_w + 15) >> 4

    def flush_effects():
        copies = [
            pltpu.async_copy(staging.at[pl.ds(h * 16, 16)],
                             out_hbm.at[plist.at[h]], sem)
            for h in range(RCAP // 16)
        ]
        for cph in copies:
            cph.wait()
        reset_plist()

    def scan_chunk(ck):
        @pl.loop(0, nv, init_carry=jnp.int32(0))
        def _scan(i, n2):
            e = idlist[pl.ds(i * 16, 16)]
            m2 = ((e >> 9) & 63) == ck
            plsc.store_compressed(cwork.at[pl.ds(n2, 16)], e, mask=m2)
            return n2 + plsc.all_reduce_population_count(m2)[0]

        return _scan

    def gather_chunk(n2, slot, sl):
        ngrp = (n2 + 15) >> 4

        @pl.loop(0, ngrp, init_carry=slot)
        def _grp(g, s):
            ev = cwork[pl.ds(g * 16, 16)]
            full = s > jnp.int32(RCAP - 16)

            @pl.when(full)
            def _():
                flush_effects()

            s = jnp.where(full, jnp.int32(0), s)
            cnt = jnp.minimum(jnp.int32(16), n2 - g * 16)

            @pl.loop(0, cnt, init_carry=s)
            def _one(i, s2):
                lane = (iota == i)
                e = jnp.sum(jnp.where(lane, ev, 0))
                cc = e & 511
                pos = e >> 15
                ccv = jnp.full((16,), cc, jnp.int32)
                for j in range(4):
                    vals = plsc.load_gather(sl, [iota + 16 * j, ccv])
                    staging[s2, pl.ds(16 * j, 16)] = vals
                row = s2 >> 4
                prow = plist[row]
                plist[row] = jnp.where(iota == (s2 & 15), pos, prow)
                return s2 + 1

            return _one

        return _grp

    def issue(k, sl, sm):
        start = jnp.minimum((k * NW + wid) * C, jnp.int32((NFULL - 1) * C))
        pltpu.async_copy(table_t_hbm.at[:, pl.ds(start, C)], sl, sm)

    def wait_slab(sl, sm):
        pltpu.make_async_copy(table_t_hbm.at[:, pl.ds(0, C)], sl, sm).wait()

    def ck_eff(k):
        return jnp.where(k * NW + wid <= NFULL - 1, k, jnp.int32(63))

    issue(jnp.int32(0), slabA, semA)

    @pl.loop(0, 31, init_carry=jnp.int32(0))
    def _main(k2, s):
        k = k2 * 2
        issue(k + 1, slabB, semB)
        n2 = scan_chunk(ck_eff(k))
        wait_slab(slabA, semA)
        s = gather_chunk(n2, s, slabA)
        issue(k + 2, slabA, semA)
        n2 = scan_chunk(ck_eff(k + 1))
        wait_slab(slabB, semB)
        s = gather_chunk(n2, s, slabB)
        return s

    wait_slab(slabA, semA)

    def _tail(slot):
        pltpu.sync_copy(tail_hbm, tailslab)
        n2 = scan_chunk(jnp.int32(61))
        return gather_chunk(n2, slot, tailslab)

    slot = lax.cond(wid == 1, _tail, lambda s: s, _main)
    del slot
    flush_effects()


BLK = 2048
GRID = B // BLK


def _tc_body(g_ref, bktt_ref, wbkt_ref, bb_ref, we_ref, wb2_ref, bc_ref,
             out_ref):
    f_t = jnp.dot(wbkt_ref[...], bktt_ref[...],
                  preferred_element_type=jnp.float32) + bb_ref[...]
    emb = g_ref[...][:, :D]
    y = lax.dot_general(we_ref[...], emb, (((1,), (1,)), ((), ())),
                        preferred_element_type=jnp.float32)
    y = y + jnp.dot(wb2_ref[...], f_t, preferred_element_type=jnp.float32)
    out_ref[...] = jnp.tanh(y + bc_ref[...])


_tc_dense = pl.pallas_call(
    _tc_body,
    grid=(GRID,),
    in_specs=[
        pl.BlockSpec((BLK, 128), lambda i: (i, 0)),
        pl.BlockSpec((4, BLK), lambda i: (0, i)),
        pl.BlockSpec((HALF, 4), lambda i: (0, 0)),
        pl.BlockSpec((HALF, 1), lambda i: (0, 0)),
        pl.BlockSpec((D, D), lambda i: (0, 0)),
        pl.BlockSpec((D, HALF), lambda i: (0, 0)),
        pl.BlockSpec((D, 1), lambda i: (0, 0)),
    ],
    out_specs=pl.BlockSpec((D, BLK), lambda i: (0, i)),
    out_shape=jax.ShapeDtypeStruct((D, B), jnp.float32),
)


def kernel(skill_ids, bkt_params, table, W_bkt, b_bkt, W_comb, b_comb):
    ids = skill_ids.astype(jnp.int32)
    table_t = table.T
    gathered_pad = _sc_stream_select(ids, table_t,
                                     table_t[:, TAIL_START:])
    out_t = _tc_dense(
        gathered_pad,
        bkt_params.T,
        W_bkt,
        b_bkt.reshape(HALF, 1),
        W_comb[:, :D],
        W_comb[:, D:],
        b_comb.reshape(D, 1),
    )
    return out_t.T


# speedup vs baseline: 1.8800x; 1.0024x over previous
import functools

import jax
import jax.numpy as jnp
from jax import lax
from jax.experimental import pallas as pl
from jax.experimental.pallas import tpu as pltpu
from jax.experimental.pallas import tpu_sc as plsc

B = 16384
D = 64
HALF = 32
NC, NS = 2, 16
NW = NC * NS
V = 1000000

C = 512
NFULL = 1953
TAIL_START = NFULL * C
TAIL_W = V - TAIL_START
RCAP = 128
DUMP = B
OUTROWS = B + 8
SENT = 0x7FFFFFFF

_mesh = plsc.VectorSubcoreMesh(core_axis_name="c", subcore_axis_name="s")


@functools.partial(
    pl.kernel,
    mesh=_mesh,
    out_type=jax.ShapeDtypeStruct((OUTROWS, 128), jnp.float32),
    scratch_types=[
        pltpu.VMEM((4096,), jnp.int32),
        pltpu.VMEM((16400,), jnp.int32),
        pltpu.VMEM((16400,), jnp.int32),
        pltpu.VMEM((D, C), jnp.float32),
        pltpu.VMEM((D, C), jnp.float32),
        pltpu.VMEM((D, TAIL_W), jnp.float32),
        pltpu.VMEM((RCAP, 128), jnp.float32),
        pltpu.VMEM((RCAP // 16, 16), jnp.int32),
        pltpu.SemaphoreType.DMA,
        pltpu.SemaphoreType.DMA,
        pltpu.SemaphoreType.DMA,
    ],
    compiler_params=pltpu.CompilerParams(use_tc_tiling_on_sc=True,
                                         needs_layout_passes=False),
)
def _sc_stream_select(ids_hbm, table_t_hbm, tail_hbm, out_hbm, idsbuf, idlist,
                      cwork, slabA, slabB, tailslab, staging, plist, sem,
                      semA, semB):
    wid = lax.axis_index("s") * NC + lax.axis_index("c")
    iota = lax.iota(jnp.int32, 16)
    sentv = jnp.full((16,), SENT, jnp.int32)
    dumpv = jnp.full((16,), jnp.int32(DUMP), jnp.int32)

    def reset_plist():
        for h in range(RCAP // 16):
            plist[h] = dumpv

    reset_plist()

    @pl.loop(0, 16400 // 16)
    def _(i):
        idlist[pl.ds(i * 16, 16)] = sentv

    off = jnp.int32(0)
    for p in range(4):
        pltpu.sync_copy(ids_hbm.at[pl.ds(p * 4096, 4096)], idsbuf)

        @pl.loop(0, 256, init_carry=off)
        def _collect(i, acc, _p=p):
            v = idsbuf[pl.ds(i * 16, 16)]
            m = ((v >> 9) & 31) == wid
            pos = iota + (_p * 4096 + i * 16)
            packed = (pos << 15) | ((v >> 14) << 9) | (v & 511)
            plsc.store_compressed(idlist.at[pl.ds(acc, 16)], packed, mask=m)
            return acc + plsc.all_reduce_population_count(m)[0]

        off = _collect

    n_w = off
    nv = (n_w + 15) >> 4

    def flush_effects():
        copies = [
            pltpu.async_copy(staging.at[pl.ds(h * 16, 16)],
                             out_hbm.at[plist.at[h]], sem)
            for h in range(RCAP // 16)
        ]
        for cph in copies:
            cph.wait()
        reset_plist()

    def scan_chunk(ck):
        @pl.loop(0, nv, init_carry=jnp.int32(0))
        def _scan(i, n2):
            e = idlist[pl.ds(i * 16, 16)]
            m2 = ((e >> 9) & 63) == ck
            plsc.store_compressed(cwork.at[pl.ds(n2, 16)], e, mask=m2)
            return n2 + plsc.all_reduce_population_count(m2)[0]

        return _scan

    def gather_chunk(n2, slot, sl):
        ngrp = (n2 + 15) >> 4

        @pl.loop(0, ngrp, init_carry=slot)
        def _grp(g, s):
            ev = cwork[pl.ds(g * 16, 16)]
            full = s > jnp.int32(RCAP - 16)

            @pl.when(full)
            def _():
                flush_effects()

            s = jnp.where(full, jnp.int32(0), s)
            cnt = jnp.minimum(jnp.int32(16), n2 - g * 16)

            @pl.loop(0, cnt, init_carry=s)
            def _one(i, s2):
                lane = (iota == i)
                e = jnp.sum(jnp.where(lane, ev, 0))
                cc = e & 511
                pos = e >> 15
                ccv = jnp.full((16,), cc, jnp.int32)
                for j in range(4):
                    vals = plsc.load_gather(sl, [iota + 16 * j, ccv])
                    staging[s2, pl.ds(16 * j, 16)] = vals
                row = s2 >> 4
                prow = plist[row]
                plist[row] = jnp.where(iota == (s2 & 15), pos, prow)
                return s2 + 1

            return _one

        return _grp

    def issue(k, sl, sm):
        start = jnp.minimum((k * NW + wid) * C, jnp.int32((NFULL - 1) * C))
        for t in range(8):
            pltpu.async_copy(
                table_t_hbm.at[pl.ds(t * 8, 8), pl.ds(start, C)],
                sl.at[pl.ds(t * 8, 8)], sm)

    def wait_slab(sl, sm):
        for t in range(8):
            pltpu.make_async_copy(
                table_t_hbm.at[pl.ds(t * 8, 8), pl.ds(0, C)],
                sl.at[pl.ds(t * 8, 8)], sm).wait()

    def ck_eff(k):
        return jnp.where(k * NW + wid <= NFULL - 1, k, jnp.int32(63))

    issue(jnp.int32(0), slabA, semA)

    @pl.loop(0, 31, init_carry=jnp.int32(0))
    def _main(k2, s):
        k = k2 * 2
        issue(k + 1, slabB, semB)
        n2 = scan_chunk(ck_eff(k))
        wait_slab(slabA, semA)
        s = gather_chunk(n2, s, slabA)
        issue(k + 2, slabA, semA)
        n2 = scan_chunk(ck_eff(k + 1))
        wait_slab(slabB, semB)
        s = gather_chunk(n2, s, slabB)
        return s

    wait_slab(slabA, semA)

    def _tail(slot):
        pltpu.sync_copy(tail_hbm, tailslab)
        n2 = scan_chunk(jnp.int32(61))
        return gather_chunk(n2, slot, tailslab)

    slot = lax.cond(wid == 1, _tail, lambda s: s, _main)
    del slot
    flush_effects()


BLK = 2048
GRID = B // BLK


def _tc_body(g_ref, bktt_ref, wbkt_ref, bb_ref, we_ref, wb2_ref, bc_ref,
             out_ref):
    f_t = jnp.dot(wbkt_ref[...], bktt_ref[...],
                  preferred_element_type=jnp.float32) + bb_ref[...]
    emb = g_ref[...][:, :D]
    y = lax.dot_general(we_ref[...], emb, (((1,), (1,)), ((), ())),
                        preferred_element_type=jnp.float32)
    y = y + jnp.dot(wb2_ref[...], f_t, preferred_element_type=jnp.float32)
    out_ref[...] = jnp.tanh(y + bc_ref[...])


_tc_dense = pl.pallas_call(
    _tc_body,
    grid=(GRID,),
    in_specs=[
        pl.BlockSpec((BLK, 128), lambda i: (i, 0)),
        pl.BlockSpec((4, BLK), lambda i: (0, i)),
        pl.BlockSpec((HALF, 4), lambda i: (0, 0)),
        pl.BlockSpec((HALF, 1), lambda i: (0, 0)),
        pl.BlockSpec((D, D), lambda i: (0, 0)),
        pl.BlockSpec((D, HALF), lambda i: (0, 0)),
        pl.BlockSpec((D, 1), lambda i: (0, 0)),
    ],
    out_specs=pl.BlockSpec((D, BLK), lambda i: (0, i)),
    out_shape=jax.ShapeDtypeStruct((D, B), jnp.float32),
)


def kernel(skill_ids, bkt_params, table, W_bkt, b_bkt, W_comb, b_comb):
    ids = skill_ids.astype(jnp.int32)
    table_t = table.T
    gathered_pad = _sc_stream_select(ids, table_t,
                                     table_t[:, TAIL_START:])
    out_t = _tc_dense(
        gathered_pad,
        bkt_params.T,
        W_bkt,
        b_bkt.reshape(HALF, 1),
        W_comb[:, :D],
        W_comb[:, D:],
        b_comb.reshape(D, 1),
    )
    return out_t.T
